# baseline (speedup 1.0000x reference)
import jax
import jax.numpy as jnp
from jax import lax
from jax.experimental import pallas as pl
from jax.experimental.pallas import tpu as pltpu
from jax.experimental.pallas import tpu_sc as plsc

NC, NS = 2, 16
NW = NC * NS
H = 50
B = 16384
D = 64
VOCAB = 1_000_000
BB_PER_W = (B // 128) // NW
NCHUNK = VOCAB // 128
CHUNK_BASE = NCHUNK // NW
CHUNK_EXTRA = NCHUNK % NW


def _detile_body(wt_hbm, wtail_hbm, tlin_hbm, s0, s1, o0, o1,
                 is0, is1, os0, os1):
    wid = lax.axis_index("s") * NC + lax.axis_index("c")
    staged = (s0, s1)
    obuf = (o0, o1)
    isem = (is0, is1)
    osem = (os0, os1)
    iota16 = lax.iota(jnp.int32, 16)
    rowv = [iota16 + 16 * j for j in range(4)]

    def cid(k):
        return lax.min(wid + NW * k, NCHUNK - 1)

    def fire_in(k, kb):
        pltpu.async_copy(
            wt_hbm.at[pl.ds(0, D), pl.ds(cid(k) * 128, 128)],
            staged[kb], isem[kb])

    def transform(sb, ob, ncols):
        def tbody(rp, c):
            lo = jnp.full((16,), 2 * rp, jnp.int32)
            hi = lo + 1
            for j in range(8):
                vals = plsc.load_gather(sb, [rowv[j % 4], lo if j < 4 else hi])
                ob[rp, pl.ds(16 * j, 16)] = vals
            return c
        lax.fori_loop(0, ncols // 2, tbody, 0)

    steps = CHUNK_BASE + 2
    fire_in(0, 0)

    def pair(p, carry):
        for kb in range(2):
            k = 2 * p + kb
            pltpu.make_async_copy(
                wt_hbm.at[pl.ds(0, D), pl.ds(0, 128)], staged[kb],
                isem[kb]).wait()
            fire_in(k + 1, 1 - kb)

            @pl.when(k >= 2)
            def _wait_out(kb=kb):
                pltpu.make_async_copy(
                    obuf[kb], tlin_hbm.at[pl.ds(0, D), pl.ds(0, 128)],
                    osem[kb]).wait()

            transform(staged[kb], obuf[kb], 128)
            pltpu.async_copy(
                obuf[kb], tlin_hbm.at[pl.ds(cid(k) * D, D), pl.ds(0, 128)],
                osem[kb])
        return carry

    lax.fori_loop(0, steps // 2, pair, 0)
    pltpu.make_async_copy(
        wt_hbm.at[pl.ds(0, D), pl.ds(0, 128)], staged[0], isem[0]).wait()
    for kb in range(2):
        pltpu.make_async_copy(
            obuf[kb], tlin_hbm.at[pl.ds(0, D), pl.ds(0, 128)],
            osem[kb]).wait()

    @pl.when(wid == NW - 1)
    def _rem():
        pltpu.sync_copy(wtail_hbm, o0.at[pl.ds(0, 32)])
        pltpu.sync_copy(o0.at[pl.ds(0, 32)],
                        tlin_hbm.at[pl.ds(NCHUNK * D, 32), pl.ds(0, 128)])


def _gather_body(idx_hbm, table_hbm, out_hbm,
                 idx_t, rowidx, g0, g1, g2, g3, o0, o1,
                 gs0, gs1, gs2, gs3, os0, os1):
    wid = lax.axis_index("s") * NC + lax.axis_index("c")
    gbuf = (g0, g1, g2, g3)
    obuf = (o0, o1)
    gsem = (gs0, gs1, gs2, gs3)
    osem = (os0, os1)
    iota16 = lax.iota(jnp.int32, 16)
    rowv = [iota16 + 16 * j for j in range(8)]

    def conv_rows(nrows):
        def conv(k, c):
            r = k // 8
            cc = 16 * (k % 8)
            v = idx_t[r, pl.ds(cc, 16)]
            rowidx[r, pl.ds(cc, 16)] = lax.shift_right_logical(v, 1)
            return c
        lax.fori_loop(0, 8 * nrows, conv, 0)

    def fire(hh):
        pltpu.async_copy(table_hbm.at[rowidx.at[hh]], gbuf[hh % 4],
                         gsem[hh % 4])

    def unit(hh, h, col0):
        gb = gbuf[hh % 4]
        ob = obuf[hh % 2]
        pltpu.make_async_copy(table_hbm.at[rowidx.at[hh]], gb,
                              gsem[hh % 4]).wait()
        par64 = []
        for j in range(8):
            iv = idx_t[hh, pl.ds(16 * j, 16)]
            par64.append(lax.shift_left(jnp.bitwise_and(iv, 1), 6))

        def tbody(dcol, c):
            for j in range(8):
                vals = plsc.load_gather(gb, [rowv[j], par64[j] + dcol])
                ob[dcol, pl.ds(16 * j, 16)] = vals
            return c

        lax.fori_loop(0, D, tbody, 0)
        pltpu.async_copy(
            ob, out_hbm.at[h, pl.ds(0, D), pl.ds(col0, 128)],
            osem[hh % 2])

    def wait_out(k, col0):
        pltpu.make_async_copy(
            obuf[k], out_hbm.at[0, pl.ds(0, D), pl.ds(col0, 128)],
            osem[k]).wait()

    def block(blk, carry):
        bbi = blk // 6
        ho = blk % 6
        col0 = (wid * BB_PER_W + bbi) * 128
        base = 8 * ho

        pltpu.sync_copy(idx_hbm.at[pl.ds(base, 8), pl.ds(col0, 128)], idx_t)
        conv_rows(8)
        for hh in range(4):
            fire(hh)
        for hh in range(8):
            if hh >= 2:
                wait_out(hh % 2, col0)
            unit(hh, base + hh, col0)
            if hh < 4:
                fire(hh + 4)
        for k in range(2):
            wait_out(k, col0)
        return carry

    lax.fori_loop(0, 6 * BB_PER_W, block, 0)

    def tail(bbi, carry):
        col0 = (wid * BB_PER_W + bbi) * 128
        pltpu.sync_copy(idx_hbm.at[pl.ds(48, 2), pl.ds(col0, 128)],
                        idx_t.at[pl.ds(0, 2)])
        conv_rows(2)
        fire(0)
        fire(1)
        unit(0, 48, col0)
        unit(1, 49, col0)
        for k in range(2):
            wait_out(k, col0)
        return carry

    lax.fori_loop(0, BB_PER_W, tail, 0)


@jax.jit
def kernel(token_ids, weight):
    idx_t = token_ids.T
    wt = weight.T
    wtail = weight[NCHUNK * 128:, :].reshape(32, 128)
    mesh = plsc.VectorSubcoreMesh(
        core_axis_name="c", subcore_axis_name="s", num_cores=NC, num_subcores=NS
    )
    params = pltpu.CompilerParams(
        use_tc_tiling_on_sc=True, needs_layout_passes=False)

    table2 = pl.kernel(
        _detile_body,
        out_type=jax.ShapeDtypeStruct((VOCAB // 2, 128), jnp.float32),
        mesh=mesh,
        scratch_types=[
            pltpu.VMEM((D, 128), jnp.float32),
            pltpu.VMEM((D, 128), jnp.float32),
            pltpu.VMEM((D, 128), jnp.float32),
            pltpu.VMEM((D, 128), jnp.float32),
            pltpu.SemaphoreType.DMA,
            pltpu.SemaphoreType.DMA,
            pltpu.SemaphoreType.DMA,
            pltpu.SemaphoreType.DMA,
        ],
        compiler_params=params,
    )(wt, wtail)

    out = pl.kernel(
        _gather_body,
        out_type=jax.ShapeDtypeStruct((H, D, B), jnp.float32),
        mesh=mesh,
        scratch_types=[
            pltpu.VMEM((8, 128), jnp.int32),
            pltpu.VMEM((8, 128), jnp.int32),
            pltpu.VMEM((128, 128), jnp.float32),
            pltpu.VMEM((128, 128), jnp.float32),
            pltpu.VMEM((128, 128), jnp.float32),
            pltpu.VMEM((128, 128), jnp.float32),
            pltpu.VMEM((D, 128), jnp.float32),
            pltpu.VMEM((D, 128), jnp.float32),
            pltpu.SemaphoreType.DMA,
            pltpu.SemaphoreType.DMA,
            pltpu.SemaphoreType.DMA,
            pltpu.SemaphoreType.DMA,
            pltpu.SemaphoreType.DMA,
            pltpu.SemaphoreType.DMA,
        ],
        compiler_params=params,
    )(idx_t, table2)
    return out.transpose(2, 0, 1)


# speedup vs baseline: 2.3735x; 2.3735x over previous
import jax
import jax.numpy as jnp
from jax import lax
from jax.experimental import pallas as pl
from jax.experimental.pallas import tpu as pltpu
from jax.experimental.pallas import tpu_sc as plsc

NUM_EMB = 1_000_000
DIM = 64
BATCH = 16384
HIST = 50
TOTAL = BATCH * HIST
RPS = 128
K = 5
CHUNK = K * RPS
NC, NS = 2, 16
NW = NC * NS
IDX_ROWS = TOTAL // RPS
ROWS_PER_W = IDX_ROWS // NW
NCHUNKS = ROWS_PER_W // K


def _emb_body(idx_hbm, table_hbm, out_hbm, idx0, idx1, rows0, rows1,
              si0, si1, sg0, sg1, so0, so1):
    wid = lax.axis_index("s") * NC + lax.axis_index("c")
    row0 = wid * ROWS_PER_W
    idx_v = (idx0, idx1)
    rows_v = (rows0, rows1)
    sem_i = (si0, si1)
    sem_g = (sg0, sg1)
    sem_o = (so0, so1)
    last = row0 + ROWS_PER_W - K

    def start_idx(g, b):
        base = lax.min(row0 + g * K, last)
        pltpu.async_copy(idx_hbm.at[pl.ds(base * RPS, CHUNK)], idx_v[b], sem_i[b])

    def do_chunk(g, b, wait_out):
        base = row0 + g * K
        pltpu.make_async_copy(idx_hbm.at[pl.ds(row0 * RPS, CHUNK)], idx_v[b],
                              sem_i[b]).wait()
        if wait_out:
            pltpu.make_async_copy(rows_v[b], out_hbm.at[pl.ds(base * RPS, CHUNK)],
                                  sem_o[b]).wait()
        pltpu.async_copy(table_hbm.at[idx_v[b]], rows_v[b], sem_g[b]).wait()
        start_idx(g + 2, b)
        pltpu.async_copy(rows_v[b], out_hbm.at[pl.ds(base * RPS, CHUNK)], sem_o[b])

    start_idx(0, 0)
    start_idx(1, 1)
    do_chunk(0, 0, False)
    do_chunk(1, 1, False)

    def pair(i, carry):
        g = 2 * i
        do_chunk(g, 0, True)
        do_chunk(g + 1, 1, True)
        return carry

    lax.fori_loop(1, NCHUNKS // 2, pair, 0)

    for b in range(2):
        pltpu.make_async_copy(rows_v[b], out_hbm.at[pl.ds(row0 * RPS, CHUNK)],
                              sem_o[b]).wait()
        pltpu.make_async_copy(idx_hbm.at[pl.ds(row0 * RPS, CHUNK)], idx_v[b],
                              sem_i[b]).wait()


@jax.jit
def kernel(token_ids, weight):
    idx = token_ids.T.reshape(TOTAL)
    mesh = plsc.VectorSubcoreMesh(
        core_axis_name="c", subcore_axis_name="s", num_cores=NC, num_subcores=NS
    )
    out = pl.kernel(
        _emb_body,
        out_type=jax.ShapeDtypeStruct((TOTAL, DIM), jnp.float32),
        mesh=mesh,
        scratch_types=[
            pltpu.VMEM((CHUNK,), jnp.int32),
            pltpu.VMEM((CHUNK,), jnp.int32),
            pltpu.VMEM((CHUNK, DIM), jnp.float32),
            pltpu.VMEM((CHUNK, DIM), jnp.float32),
            pltpu.SemaphoreType.DMA,
            pltpu.SemaphoreType.DMA,
            pltpu.SemaphoreType.DMA,
            pltpu.SemaphoreType.DMA,
            pltpu.SemaphoreType.DMA,
            pltpu.SemaphoreType.DMA,
        ],
        compiler_params=pltpu.CompilerParams(use_tc_tiling_on_sc=False),
    )(idx, weight)
    return out.reshape(HIST, BATCH, DIM).transpose(1, 0, 2)
